# Initial kernel scaffold; baseline (speedup 1.0000x reference)
import functools

import jax
import jax.numpy as jnp
from jax import lax
from jax.experimental import pallas as pl
from jax.experimental.pallas import tpu as pltpu
from jax.experimental.pallas import tpu_sc as plsc

N = 10000
C = 128
K = 27
EK = 12000
E = K * EK
EPS = 1e-5

NUM_CORES = 2
NUM_SUBCORES = 16
WIN = 120
NWIN = E // WIN
NWIN_PER_CORE = NWIN // NUM_CORES
MAX_WIN_PER_SUB = -(-NWIN_PER_CORE // NUM_SUBCORES)
ROWS_PER_SUB = N // NUM_SUBCORES

MM_BLK = 1000
NB = N // MM_BLK


def _mm_body(x_ref, w_ref, z_ref):
    z_ref[0] = jnp.dot(x_ref[...], w_ref[0],
                       preferred_element_type=jnp.float32,
                       precision=lax.Precision.HIGHEST)


def _z_from_x(x, w):
    return pl.pallas_call(
        _mm_body,
        grid=(NB, K),
        in_specs=[
            pl.BlockSpec((MM_BLK, C), lambda i, k: (i, 0)),
            pl.BlockSpec((1, C, C), lambda i, k: (k, 0, 0)),
        ],
        out_specs=pl.BlockSpec((1, MM_BLK, C), lambda i, k: (k, i, 0)),
        out_shape=jax.ShapeDtypeStruct((K, N, C), jnp.float32),
    )(x, w)


def _mm_norm_body(p_ref, sc_ref, sh_ref, w_ref, z_ref):
    h = (p_ref[0] + p_ref[1]) * sc_ref[...] + sh_ref[...]
    h = jnp.maximum(h, 0.0)
    z_ref[0] = jnp.dot(h, w_ref[0],
                       preferred_element_type=jnp.float32,
                       precision=lax.Precision.HIGHEST)


def _z_from_partials(p, scale, shift, w):
    return pl.pallas_call(
        _mm_norm_body,
        grid=(NB, K),
        in_specs=[
            pl.BlockSpec((NUM_CORES, MM_BLK, C), lambda i, k: (0, i, 0)),
            pl.BlockSpec((1, C), lambda i, k: (0, 0)),
            pl.BlockSpec((1, C), lambda i, k: (0, 0)),
            pl.BlockSpec((1, C, C), lambda i, k: (k, 0, 0)),
        ],
        out_specs=pl.BlockSpec((1, MM_BLK, C), lambda i, k: (k, i, 0)),
        out_shape=jax.ShapeDtypeStruct((K, N, C), jnp.float32),
    )(p, scale, shift, w)


def _stats_body(p_ref, sum_ref, sq_ref):
    i = pl.program_id(0)

    @pl.when(i == 0)
    def _():
        sum_ref[...] = jnp.zeros_like(sum_ref)
        sq_ref[...] = jnp.zeros_like(sq_ref)

    h = p_ref[0] + p_ref[1]
    sum_ref[...] += jnp.sum(h, axis=0, keepdims=True)
    sq_ref[...] += jnp.sum(h * h, axis=0, keepdims=True)


def _stats(p):
    return pl.pallas_call(
        _stats_body,
        grid=(NB,),
        in_specs=[pl.BlockSpec((NUM_CORES, MM_BLK, C), lambda i: (0, i, 0))],
        out_specs=[
            pl.BlockSpec((1, C), lambda i: (0, 0)),
            pl.BlockSpec((1, C), lambda i: (0, 0)),
        ],
        out_shape=[
            jax.ShapeDtypeStruct((1, C), jnp.float32),
            jax.ShapeDtypeStruct((1, C), jnp.float32),
        ],
    )(p)


def _final_body(p_ref, sc_ref, sh_ref, x_ref, o_ref):
    h = (p_ref[0] + p_ref[1]) * sc_ref[...] + sh_ref[...] + x_ref[...]
    o_ref[...] = jnp.maximum(h, 0.0)


def _final(p, scale, shift, x):
    return pl.pallas_call(
        _final_body,
        grid=(NB,),
        in_specs=[
            pl.BlockSpec((NUM_CORES, MM_BLK, C), lambda i: (0, i, 0)),
            pl.BlockSpec((1, C), lambda i: (0, 0)),
            pl.BlockSpec((1, C), lambda i: (0, 0)),
            pl.BlockSpec((MM_BLK, C), lambda i: (i, 0)),
        ],
        out_specs=pl.BlockSpec((MM_BLK, C), lambda i: (i, 0)),
        out_shape=jax.ShapeDtypeStruct((N, C), jnp.float32),
    )(p, scale, shift, x)


@functools.partial(
    pl.kernel,
    out_type=jax.ShapeDtypeStruct((NUM_CORES, N, C), jnp.float32),
    mesh=plsc.VectorSubcoreMesh(core_axis_name="c", subcore_axis_name="s"),
    scratch_types=[
        pltpu.VMEM((WIN,), jnp.int32),
        pltpu.VMEM((WIN,), jnp.int32),
        pltpu.VMEM((WIN, C), jnp.float32),
        pltpu.VMEM_SHARED((N, C), jnp.float32),
        pltpu.SemaphoreType.DMA,
    ],
)
def _edge_conv(z_hbm, srcf_hbm, dst_hbm, zero_hbm, out_hbm,
               srcv, dstv, rows, acc, sem):
    c = lax.axis_index("c")
    s = lax.axis_index("s")

    pltpu.sync_copy(zero_hbm.at[pl.ds(s * ROWS_PER_SUB, ROWS_PER_SUB)],
                    acc.at[pl.ds(s * ROWS_PER_SUB, ROWS_PER_SUB)])
    plsc.subcore_barrier()

    @pl.loop(0, MAX_WIN_PER_SUB)
    def _(j):
        win = s + NUM_SUBCORES * j

        @pl.when(win < NWIN_PER_CORE)
        def _():
            off = (c * NWIN_PER_CORE + win) * WIN
            pltpu.sync_copy(srcf_hbm.at[pl.ds(off, WIN)], srcv)
            pltpu.sync_copy(dst_hbm.at[pl.ds(off, WIN)], dstv)
            pltpu.async_copy(z_hbm.at[srcv], rows, sem).wait()
            pltpu.sync_copy(rows, acc.at[dstv], add=True)

    plsc.subcore_barrier()
    pltpu.sync_copy(acc.at[pl.ds(s * ROWS_PER_SUB, ROWS_PER_SUB)],
                    out_hbm.at[c].at[pl.ds(s * ROWS_PER_SUB, ROWS_PER_SUB)])


def _bn_coeffs(s, q, g, b):
    mu = s / N
    var = q / N - mu * mu
    rs = g.reshape(1, C) / jnp.sqrt(var + EPS)
    return rs, b.reshape(1, C) - mu * rs


def kernel(x, edge_index, W1, g1, b1, W2, g2, b2):
    koff = (jnp.arange(E, dtype=jnp.int32) // EK) * N
    srcf = edge_index[0] + koff
    dst = edge_index[1]
    zeros = jnp.zeros((N, C), jnp.float32)

    z1 = _z_from_x(x, W1).reshape(K * N, C)
    p1 = _edge_conv(z1, srcf, dst, zeros)
    s1, q1 = _stats(p1)
    scale1, shift1 = _bn_coeffs(s1, q1, g1, b1)

    z2 = _z_from_partials(p1, scale1, shift1, W2).reshape(K * N, C)
    p2 = _edge_conv(z2, srcf, dst, zeros)
    s2, q2 = _stats(p2)
    scale2, shift2 = _bn_coeffs(s2, q2, g2, b2)

    return _final(p2, scale2, shift2, x)


# speedup vs baseline: 2.8647x; 2.8647x over previous
import functools

import jax
import jax.numpy as jnp
from jax import lax
from jax.experimental import pallas as pl
from jax.experimental.pallas import tpu as pltpu
from jax.experimental.pallas import tpu_sc as plsc

N = 10000
C = 128
K = 27
EK = 12000
E = K * EK
EPS = 1e-5

NUM_CORES = 2
NUM_SUBCORES = 16
WIN = 120
NWIN = E // WIN
NWIN_PER_CORE = NWIN // NUM_CORES
MAX_WIN_PER_SUB = -(-NWIN_PER_CORE // NUM_SUBCORES)
ROWS_UNIF = 624
TAIL_OFF = ROWS_UNIF * NUM_SUBCORES
TAIL = N - TAIL_OFF

MM_BLK = 1000
NB = N // MM_BLK


def _mm_body(x_ref, w_ref, z_ref):
    z_ref[0] = jnp.dot(x_ref[...], w_ref[0],
                       preferred_element_type=jnp.float32,
                       precision=lax.Precision.HIGHEST)


def _z_from_x(x, w):
    return pl.pallas_call(
        _mm_body,
        grid=(NB, K),
        in_specs=[
            pl.BlockSpec((MM_BLK, C), lambda i, k: (i, 0)),
            pl.BlockSpec((1, C, C), lambda i, k: (k, 0, 0)),
        ],
        out_specs=pl.BlockSpec((1, MM_BLK, C), lambda i, k: (k, i, 0)),
        out_shape=jax.ShapeDtypeStruct((K, N, C), jnp.float32),
    )(x, w)


def _mm_norm_body(p_ref, sc_ref, sh_ref, w_ref, z_ref):
    h = (p_ref[0] + p_ref[1]) * sc_ref[...] + sh_ref[...]
    h = jnp.maximum(h, 0.0)
    z_ref[0] = jnp.dot(h, w_ref[0],
                       preferred_element_type=jnp.float32,
                       precision=lax.Precision.HIGHEST)


def _z_from_partials(p, scale, shift, w):
    return pl.pallas_call(
        _mm_norm_body,
        grid=(NB, K),
        in_specs=[
            pl.BlockSpec((NUM_CORES, MM_BLK, C), lambda i, k: (0, i, 0)),
            pl.BlockSpec((1, C), lambda i, k: (0, 0)),
            pl.BlockSpec((1, C), lambda i, k: (0, 0)),
            pl.BlockSpec((1, C, C), lambda i, k: (k, 0, 0)),
        ],
        out_specs=pl.BlockSpec((1, MM_BLK, C), lambda i, k: (k, i, 0)),
        out_shape=jax.ShapeDtypeStruct((K, N, C), jnp.float32),
    )(p, scale, shift, w)


def _stats_body(p_ref, sum_ref, sq_ref):
    i = pl.program_id(0)

    @pl.when(i == 0)
    def _():
        sum_ref[...] = jnp.zeros_like(sum_ref)
        sq_ref[...] = jnp.zeros_like(sq_ref)

    h = p_ref[0] + p_ref[1]
    sum_ref[...] += jnp.sum(h, axis=0, keepdims=True)
    sq_ref[...] += jnp.sum(h * h, axis=0, keepdims=True)


def _stats(p):
    return pl.pallas_call(
        _stats_body,
        grid=(NB,),
        in_specs=[pl.BlockSpec((NUM_CORES, MM_BLK, C), lambda i: (0, i, 0))],
        out_specs=[
            pl.BlockSpec((1, C), lambda i: (0, 0)),
            pl.BlockSpec((1, C), lambda i: (0, 0)),
        ],
        out_shape=[
            jax.ShapeDtypeStruct((1, C), jnp.float32),
            jax.ShapeDtypeStruct((1, C), jnp.float32),
        ],
    )(p)


def _final_body(p_ref, sc_ref, sh_ref, x_ref, o_ref):
    h = (p_ref[0] + p_ref[1]) * sc_ref[...] + sh_ref[...] + x_ref[...]
    o_ref[...] = jnp.maximum(h, 0.0)


def _final(p, scale, shift, x):
    return pl.pallas_call(
        _final_body,
        grid=(NB,),
        in_specs=[
            pl.BlockSpec((NUM_CORES, MM_BLK, C), lambda i: (0, i, 0)),
            pl.BlockSpec((1, C), lambda i: (0, 0)),
            pl.BlockSpec((1, C), lambda i: (0, 0)),
            pl.BlockSpec((MM_BLK, C), lambda i: (i, 0)),
        ],
        out_specs=pl.BlockSpec((MM_BLK, C), lambda i: (i, 0)),
        out_shape=jax.ShapeDtypeStruct((N, C), jnp.float32),
    )(p, scale, shift, x)


@functools.partial(
    pl.kernel,
    out_type=jax.ShapeDtypeStruct((NUM_CORES, N, C), jnp.float32),
    mesh=plsc.VectorSubcoreMesh(core_axis_name="c", subcore_axis_name="s"),
    scratch_types=[
        pltpu.VMEM((WIN,), jnp.int32),
        pltpu.VMEM((WIN,), jnp.int32),
        pltpu.VMEM((WIN, C), jnp.float32),
        pltpu.VMEM_SHARED((N, C), jnp.float32),
        pltpu.SemaphoreType.DMA,
    ],
)
def _edge_conv(z_hbm, srcf_hbm, dst_hbm, zero_hbm, out_hbm,
               srcv, dstv, rows, acc, sem):
    c = lax.axis_index("c")
    s = lax.axis_index("s")

    pltpu.sync_copy(zero_hbm.at[pl.ds(s * ROWS_UNIF, ROWS_UNIF)],
                    acc.at[pl.ds(s * ROWS_UNIF, ROWS_UNIF)])

    @pl.when(s == 0)
    def _():
        pltpu.sync_copy(zero_hbm.at[pl.ds(TAIL_OFF, TAIL)],
                        acc.at[pl.ds(TAIL_OFF, TAIL)])

    plsc.subcore_barrier()

    @pl.loop(0, MAX_WIN_PER_SUB)
    def _(j):
        win = s + NUM_SUBCORES * j

        @pl.when(win < NWIN_PER_CORE)
        def _():
            off = (c * NWIN_PER_CORE + win) * WIN
            pltpu.sync_copy(srcf_hbm.at[pl.ds(off, WIN)], srcv)
            pltpu.sync_copy(dst_hbm.at[pl.ds(off, WIN)], dstv)
            pltpu.async_copy(z_hbm.at[srcv], rows, sem).wait()
            pltpu.sync_copy(rows, acc.at[dstv], add=True)

    plsc.subcore_barrier()
    pltpu.sync_copy(acc.at[pl.ds(s * ROWS_UNIF, ROWS_UNIF)],
                    out_hbm.at[c].at[pl.ds(s * ROWS_UNIF, ROWS_UNIF)])

    @pl.when(s == 0)
    def _():
        pltpu.sync_copy(acc.at[pl.ds(TAIL_OFF, TAIL)],
                        out_hbm.at[c].at[pl.ds(TAIL_OFF, TAIL)])


def _bn_coeffs(s, q, g, b):
    mu = s / N
    var = q / N - mu * mu
    rs = g.reshape(1, C) / jnp.sqrt(var + EPS)
    return rs, b.reshape(1, C) - mu * rs


def kernel(x, edge_index, W1, g1, b1, W2, g2, b2):
    koff = (jnp.arange(E, dtype=jnp.int32) // EK) * N
    srcf = edge_index[0] + koff
    dst = edge_index[1]
    zeros = jnp.zeros((N, C), jnp.float32)

    z1 = _z_from_x(x, W1).reshape(K * N, C)
    p1 = _edge_conv(z1, srcf, dst, zeros)
    s1, q1 = _stats(p1)
    scale1, shift1 = _bn_coeffs(s1, q1, g1, b1)

    z2 = _z_from_partials(p1, scale1, shift1, W2).reshape(K * N, C)
    p2 = _edge_conv(z2, srcf, dst, zeros)
    s2, q2 = _stats(p2)
    scale2, shift2 = _bn_coeffs(s2, q2, g2, b2)

    return _final(p2, scale2, shift2, x)


# speedup vs baseline: 5.1369x; 1.7932x over previous
import functools

import jax
import jax.numpy as jnp
from jax import lax
from jax.experimental import pallas as pl
from jax.experimental.pallas import tpu as pltpu
from jax.experimental.pallas import tpu_sc as plsc

N = 10000
C = 128
K = 27
EK = 12000
E = K * EK
EPS = 1e-5

NUM_CORES = 2
NUM_SUBCORES = 16
WIN = 120
NWIN = E // WIN
NWIN_PER_CORE = NWIN // NUM_CORES
MAX_WIN_PER_SUB = -(-NWIN_PER_CORE // NUM_SUBCORES)
ROWS_UNIF = 624
TAIL_OFF = ROWS_UNIF * NUM_SUBCORES
TAIL = N - TAIL_OFF

MM_BLK = 1000
NB = N // MM_BLK


def _mm_body(x_ref, w_ref, z_ref):
    z_ref[...] = jnp.dot(x_ref[...], w_ref[...],
                         preferred_element_type=jnp.float32)


def _z_from_x(x, wr):
    return pl.pallas_call(
        _mm_body,
        grid=(NB,),
        in_specs=[
            pl.BlockSpec((MM_BLK, C), lambda i: (i, 0)),
            pl.BlockSpec((C, K * C), lambda i: (0, 0)),
        ],
        out_specs=pl.BlockSpec((MM_BLK, K * C), lambda i: (i, 0)),
        out_shape=jax.ShapeDtypeStruct((N, K * C), jnp.float32),
    )(x, wr)


def _mm_norm_body(p_ref, sc_ref, sh_ref, w_ref, z_ref):
    h = (p_ref[0] + p_ref[1]) * sc_ref[...] + sh_ref[...]
    h = jnp.maximum(h, 0.0)
    z_ref[...] = jnp.dot(h, w_ref[...],
                         preferred_element_type=jnp.float32)


def _z_from_partials(p, scale, shift, wr):
    return pl.pallas_call(
        _mm_norm_body,
        grid=(NB,),
        in_specs=[
            pl.BlockSpec((NUM_CORES, MM_BLK, C), lambda i: (0, i, 0)),
            pl.BlockSpec((1, C), lambda i: (0, 0)),
            pl.BlockSpec((1, C), lambda i: (0, 0)),
            pl.BlockSpec((C, K * C), lambda i: (0, 0)),
        ],
        out_specs=pl.BlockSpec((MM_BLK, K * C), lambda i: (i, 0)),
        out_shape=jax.ShapeDtypeStruct((N, K * C), jnp.float32),
    )(p, scale, shift, wr)


def _stats_body(p_ref, sum_ref, sq_ref):
    i = pl.program_id(0)

    @pl.when(i == 0)
    def _():
        sum_ref[...] = jnp.zeros_like(sum_ref)
        sq_ref[...] = jnp.zeros_like(sq_ref)

    h = p_ref[0] + p_ref[1]
    sum_ref[...] += jnp.sum(h, axis=0, keepdims=True)
    sq_ref[...] += jnp.sum(h * h, axis=0, keepdims=True)


def _stats(p):
    return pl.pallas_call(
        _stats_body,
        grid=(NB,),
        in_specs=[pl.BlockSpec((NUM_CORES, MM_BLK, C), lambda i: (0, i, 0))],
        out_specs=[
            pl.BlockSpec((1, C), lambda i: (0, 0)),
            pl.BlockSpec((1, C), lambda i: (0, 0)),
        ],
        out_shape=[
            jax.ShapeDtypeStruct((1, C), jnp.float32),
            jax.ShapeDtypeStruct((1, C), jnp.float32),
        ],
    )(p)


def _final_body(p_ref, sc_ref, sh_ref, x_ref, o_ref):
    h = (p_ref[0] + p_ref[1]) * sc_ref[...] + sh_ref[...] + x_ref[...]
    o_ref[...] = jnp.maximum(h, 0.0)


def _final(p, scale, shift, x):
    return pl.pallas_call(
        _final_body,
        grid=(NB,),
        in_specs=[
            pl.BlockSpec((NUM_CORES, MM_BLK, C), lambda i: (0, i, 0)),
            pl.BlockSpec((1, C), lambda i: (0, 0)),
            pl.BlockSpec((1, C), lambda i: (0, 0)),
            pl.BlockSpec((MM_BLK, C), lambda i: (i, 0)),
        ],
        out_specs=pl.BlockSpec((MM_BLK, C), lambda i: (i, 0)),
        out_shape=jax.ShapeDtypeStruct((N, C), jnp.float32),
    )(p, scale, shift, x)


NG = (MAX_WIN_PER_SUB + 1) // 2
SRC_LOC = MAX_WIN_PER_SUB * WIN


@functools.partial(
    pl.kernel,
    out_type=jax.ShapeDtypeStruct((NUM_CORES, N, C), jnp.float32),
    mesh=plsc.VectorSubcoreMesh(core_axis_name="c", subcore_axis_name="s"),
    scratch_types=[
        pltpu.VMEM((SRC_LOC,), jnp.int32),
        pltpu.VMEM((WIN,), jnp.int32),
        pltpu.VMEM((WIN,), jnp.int32),
        pltpu.VMEM((WIN, C), jnp.float32),
        pltpu.VMEM((WIN, C), jnp.float32),
        pltpu.VMEM_SHARED((N, C), jnp.float32),
        pltpu.SemaphoreType.DMA,
        pltpu.SemaphoreType.DMA,
        pltpu.SemaphoreType.DMA,
        pltpu.SemaphoreType.DMA,
    ],
)
def _edge_conv(z_hbm, srcf_hbm, dst_hbm, zero_hbm, out_hbm,
               src_loc, dstv0, dstv1, rows0, rows1, acc,
               isem0, isem1, gsem0, gsem1):
    c = lax.axis_index("c")
    s = lax.axis_index("s")

    start_win = c * NWIN_PER_CORE + s * 84 + jnp.minimum(s, 6)
    cnt = jnp.where(s < 6, 85, 84)

    pltpu.sync_copy(srcf_hbm.at[pl.ds(start_win * WIN, SRC_LOC)], src_loc)

    pltpu.sync_copy(zero_hbm.at[pl.ds(s * ROWS_UNIF, ROWS_UNIF)],
                    acc.at[pl.ds(s * ROWS_UNIF, ROWS_UNIF)])

    @pl.when(s == 0)
    def _():
        pltpu.sync_copy(zero_hbm.at[pl.ds(TAIL_OFF, TAIL)],
                        acc.at[pl.ds(TAIL_OFF, TAIL)])

    plsc.subcore_barrier()

    def fire(dstv, rows, isem, gsem, j):
        @pl.when(j < cnt)
        def _():
            pltpu.async_copy(dst_hbm.at[pl.ds((start_win + j) * WIN, WIN)],
                             dstv, isem)
            pltpu.async_copy(z_hbm.at[src_loc.at[pl.ds(j * WIN, WIN)]],
                             rows, gsem)

    def drain(dstv, rows, isem, gsem, j):
        @pl.when(j < cnt)
        def _():
            pltpu.make_async_copy(dst_hbm.at[pl.ds((start_win + j) * WIN, WIN)],
                                  dstv, isem).wait()
            pltpu.make_async_copy(z_hbm.at[src_loc.at[pl.ds(j * WIN, WIN)]],
                                  rows, gsem).wait()
            pltpu.sync_copy(rows, acc.at[dstv], add=True)

    fire(dstv0, rows0, isem0, gsem0, 0)

    @pl.loop(0, NG)
    def _(g):
        j0 = 2 * g
        fire(dstv1, rows1, isem1, gsem1, j0 + 1)
        drain(dstv0, rows0, isem0, gsem0, j0)
        fire(dstv0, rows0, isem0, gsem0, j0 + 2)
        drain(dstv1, rows1, isem1, gsem1, j0 + 1)

    plsc.subcore_barrier()
    pltpu.sync_copy(acc.at[pl.ds(s * ROWS_UNIF, ROWS_UNIF)],
                    out_hbm.at[c].at[pl.ds(s * ROWS_UNIF, ROWS_UNIF)])

    @pl.when(s == 0)
    def _():
        pltpu.sync_copy(acc.at[pl.ds(TAIL_OFF, TAIL)],
                        out_hbm.at[c].at[pl.ds(TAIL_OFF, TAIL)])


def _bn_coeffs(s, q, g, b):
    mu = s / N
    var = q / N - mu * mu
    rs = g.reshape(1, C) / jnp.sqrt(var + EPS)
    return rs, b.reshape(1, C) - mu * rs


def kernel(x, edge_index, W1, g1, b1, W2, g2, b2):
    kid = jnp.arange(E, dtype=jnp.int32) // EK
    srcf = edge_index[0] * K + kid
    srcf = jnp.concatenate([srcf, jnp.zeros((WIN,), jnp.int32)])
    dst = edge_index[1]
    zeros = jnp.zeros((N, C), jnp.float32)
    w1r = jnp.transpose(W1, (1, 0, 2)).reshape(C, K * C)
    w2r = jnp.transpose(W2, (1, 0, 2)).reshape(C, K * C)

    z1 = _z_from_x(x, w1r).reshape(N * K, C)
    p1 = _edge_conv(z1, srcf, dst, zeros)
    s1, q1 = _stats(p1)
    scale1, shift1 = _bn_coeffs(s1, q1, g1, b1)

    z2 = _z_from_partials(p1, scale1, shift1, w2r).reshape(N * K, C)
    p2 = _edge_conv(z2, srcf, dst, zeros)
    s2, q2 = _stats(p2)
    scale2, shift2 = _bn_coeffs(s2, q2, g2, b2)

    return _final(p2, scale2, shift2, x)


# speedup vs baseline: 6.9372x; 1.3505x over previous
import functools

import jax
import jax.numpy as jnp
from jax import lax
from jax.experimental import pallas as pl
from jax.experimental.pallas import tpu as pltpu
from jax.experimental.pallas import tpu_sc as plsc

N = 10000
C = 128
K = 27
EK = 12000
E = K * EK
EPS = 1e-5

NUM_CORES = 2
NUM_SUBCORES = 16
WIN = 120
NWIN = E // WIN
NWIN_PER_CORE = NWIN // NUM_CORES
MAX_WIN_PER_SUB = -(-NWIN_PER_CORE // NUM_SUBCORES)
ROWS_UNIF = 624
TAIL_OFF = ROWS_UNIF * NUM_SUBCORES
TAIL = N - TAIL_OFF

MM_BLK = 1000
NB = N // MM_BLK
KG = 3
NKG = K // KG


def _mm_body(x_ref, w_ref, z_ref):
    res = jnp.dot(x_ref[...], w_ref[...], preferred_element_type=jnp.float32)
    for t in range(KG):
        z_ref[t] = res[:, t * C:(t + 1) * C]


def _z_from_x(x, wr):
    return pl.pallas_call(
        _mm_body,
        grid=(NB, NKG),
        in_specs=[
            pl.BlockSpec((MM_BLK, C), lambda i, g: (i, 0)),
            pl.BlockSpec((C, KG * C), lambda i, g: (0, g)),
        ],
        out_specs=pl.BlockSpec((KG, MM_BLK, C), lambda i, g: (g, i, 0)),
        out_shape=jax.ShapeDtypeStruct((K, N, C), jnp.float32),
    )(x, wr)


def _mm_norm_body(p_ref, sc_ref, sh_ref, w_ref, z_ref):
    h = (p_ref[0] + p_ref[1]) * sc_ref[...] + sh_ref[...]
    h = jnp.maximum(h, 0.0)
    res = jnp.dot(h, w_ref[...], preferred_element_type=jnp.float32)
    for t in range(KG):
        z_ref[t] = res[:, t * C:(t + 1) * C]


def _z_from_partials(p, scale, shift, wr):
    return pl.pallas_call(
        _mm_norm_body,
        grid=(NB, NKG),
        in_specs=[
            pl.BlockSpec((NUM_CORES, MM_BLK, C), lambda i, g: (0, i, 0)),
            pl.BlockSpec((1, C), lambda i, g: (0, 0)),
            pl.BlockSpec((1, C), lambda i, g: (0, 0)),
            pl.BlockSpec((C, KG * C), lambda i, g: (0, g)),
        ],
        out_specs=pl.BlockSpec((KG, MM_BLK, C), lambda i, g: (g, i, 0)),
        out_shape=jax.ShapeDtypeStruct((K, N, C), jnp.float32),
    )(p, scale, shift, wr)


def _stats_body(p_ref, sum_ref, sq_ref):
    i = pl.program_id(0)

    @pl.when(i == 0)
    def _():
        sum_ref[...] = jnp.zeros_like(sum_ref)
        sq_ref[...] = jnp.zeros_like(sq_ref)

    h = p_ref[0] + p_ref[1]
    sum_ref[...] += jnp.sum(h, axis=0, keepdims=True)
    sq_ref[...] += jnp.sum(h * h, axis=0, keepdims=True)


def _stats(p):
    return pl.pallas_call(
        _stats_body,
        grid=(NB,),
        in_specs=[pl.BlockSpec((NUM_CORES, MM_BLK, C), lambda i: (0, i, 0))],
        out_specs=[
            pl.BlockSpec((1, C), lambda i: (0, 0)),
            pl.BlockSpec((1, C), lambda i: (0, 0)),
        ],
        out_shape=[
            jax.ShapeDtypeStruct((1, C), jnp.float32),
            jax.ShapeDtypeStruct((1, C), jnp.float32),
        ],
    )(p)


def _final_body(p_ref, sc_ref, sh_ref, x_ref, o_ref):
    h = (p_ref[0] + p_ref[1]) * sc_ref[...] + sh_ref[...] + x_ref[...]
    o_ref[...] = jnp.maximum(h, 0.0)


def _final(p, scale, shift, x):
    return pl.pallas_call(
        _final_body,
        grid=(NB,),
        in_specs=[
            pl.BlockSpec((NUM_CORES, MM_BLK, C), lambda i: (0, i, 0)),
            pl.BlockSpec((1, C), lambda i: (0, 0)),
            pl.BlockSpec((1, C), lambda i: (0, 0)),
            pl.BlockSpec((MM_BLK, C), lambda i: (i, 0)),
        ],
        out_specs=pl.BlockSpec((MM_BLK, C), lambda i: (i, 0)),
        out_shape=jax.ShapeDtypeStruct((N, C), jnp.float32),
    )(p, scale, shift, x)


NG = (MAX_WIN_PER_SUB + 1) // 2
SRC_LOC = MAX_WIN_PER_SUB * WIN


@functools.partial(
    pl.kernel,
    out_type=jax.ShapeDtypeStruct((NUM_CORES, N, C), jnp.float32),
    mesh=plsc.VectorSubcoreMesh(core_axis_name="c", subcore_axis_name="s"),
    scratch_types=[
        pltpu.VMEM((SRC_LOC,), jnp.int32),
        pltpu.VMEM((WIN,), jnp.int32),
        pltpu.VMEM((WIN,), jnp.int32),
        pltpu.VMEM((WIN, C), jnp.float32),
        pltpu.VMEM((WIN, C), jnp.float32),
        pltpu.VMEM_SHARED((N, C), jnp.float32),
        pltpu.SemaphoreType.DMA,
        pltpu.SemaphoreType.DMA,
        pltpu.SemaphoreType.DMA,
        pltpu.SemaphoreType.DMA,
        pltpu.SemaphoreType.DMA,
        pltpu.SemaphoreType.DMA,
    ],
)
def _edge_conv(z_hbm, srcf_hbm, dst_hbm, zero_hbm, out_hbm,
               src_loc, dstv0, dstv1, rows0, rows1, acc,
               isem0, isem1, gsem0, gsem1, ssem0, ssem1):
    c = lax.axis_index("c")
    s = lax.axis_index("s")

    start_win = c * NWIN_PER_CORE + s * 84 + jnp.minimum(s, 6)
    cnt = jnp.where(s < 6, 85, 84)

    pltpu.sync_copy(srcf_hbm.at[pl.ds(start_win * WIN, SRC_LOC)], src_loc)

    pltpu.sync_copy(zero_hbm.at[pl.ds(s * ROWS_UNIF, ROWS_UNIF)],
                    acc.at[pl.ds(s * ROWS_UNIF, ROWS_UNIF)])

    @pl.when(s == 0)
    def _():
        pltpu.sync_copy(zero_hbm.at[pl.ds(TAIL_OFF, TAIL)],
                        acc.at[pl.ds(TAIL_OFF, TAIL)])

    plsc.subcore_barrier()

    def fire(dstv, rows, isem, gsem, j):
        @pl.when(j < cnt)
        def _():
            pltpu.async_copy(dst_hbm.at[pl.ds((start_win + j) * WIN, WIN)],
                             dstv, isem)
            pltpu.async_copy(z_hbm.at[src_loc.at[pl.ds(j * WIN, WIN)]],
                             rows, gsem)

    def fire_next(dstv, rows, isem, gsem, ssem, j):
        @pl.when(j < cnt)
        def _():
            pltpu.make_async_copy(rows, acc.at[dstv], ssem).wait()
            pltpu.async_copy(dst_hbm.at[pl.ds((start_win + j) * WIN, WIN)],
                             dstv, isem)
            pltpu.async_copy(z_hbm.at[src_loc.at[pl.ds(j * WIN, WIN)]],
                             rows, gsem)

    def drain(dstv, rows, isem, gsem, ssem, j):
        @pl.when(j < cnt)
        def _():
            pltpu.make_async_copy(dst_hbm.at[pl.ds((start_win + j) * WIN, WIN)],
                                  dstv, isem).wait()
            pltpu.make_async_copy(z_hbm.at[src_loc.at[pl.ds(j * WIN, WIN)]],
                                  rows, gsem).wait()
            pltpu.async_copy(rows, acc.at[dstv], ssem, add=True)

    fire(dstv0, rows0, isem0, gsem0, 0)
    fire(dstv1, rows1, isem1, gsem1, 1)

    @pl.loop(0, NG)
    def _(g):
        j0 = 2 * g
        drain(dstv0, rows0, isem0, gsem0, ssem0, j0)
        fire_next(dstv0, rows0, isem0, gsem0, ssem0, j0 + 2)
        drain(dstv1, rows1, isem1, gsem1, ssem1, j0 + 1)
        fire_next(dstv1, rows1, isem1, gsem1, ssem1, j0 + 3)

    pltpu.make_async_copy(rows0, acc.at[dstv0], ssem0).wait()
    pltpu.make_async_copy(rows1, acc.at[dstv1], ssem1).wait()

    plsc.subcore_barrier()
    pltpu.sync_copy(acc.at[pl.ds(s * ROWS_UNIF, ROWS_UNIF)],
                    out_hbm.at[c].at[pl.ds(s * ROWS_UNIF, ROWS_UNIF)])

    @pl.when(s == 0)
    def _():
        pltpu.sync_copy(acc.at[pl.ds(TAIL_OFF, TAIL)],
                        out_hbm.at[c].at[pl.ds(TAIL_OFF, TAIL)])


def _bn_coeffs(s, q, g, b):
    mu = s / N
    var = q / N - mu * mu
    rs = g.reshape(1, C) / jnp.sqrt(var + EPS)
    return rs, b.reshape(1, C) - mu * rs


def kernel(x, edge_index, W1, g1, b1, W2, g2, b2):
    kid = jnp.arange(E, dtype=jnp.int32) // EK
    srcf = edge_index[0] + kid * N
    srcf = jnp.concatenate([srcf, jnp.zeros((WIN,), jnp.int32)])
    dst = edge_index[1]
    zeros = jnp.zeros((N, C), jnp.float32)
    w1r = jnp.transpose(W1, (1, 0, 2)).reshape(C, K * C)
    w2r = jnp.transpose(W2, (1, 0, 2)).reshape(C, K * C)

    z1 = _z_from_x(x, w1r).reshape(K * N, C)
    p1 = _edge_conv(z1, srcf, dst, zeros)
    s1, q1 = _stats(p1)
    scale1, shift1 = _bn_coeffs(s1, q1, g1, b1)

    z2 = _z_from_partials(p1, scale1, shift1, w2r).reshape(K * N, C)
    p2 = _edge_conv(z2, srcf, dst, zeros)
    s2, q2 = _stats(p2)
    scale2, shift2 = _bn_coeffs(s2, q2, g2, b2)

    return _final(p2, scale2, shift2, x)


# speedup vs baseline: 8.1253x; 1.1713x over previous
import functools

import jax
import jax.numpy as jnp
from jax import lax
from jax.experimental import pallas as pl
from jax.experimental.pallas import tpu as pltpu
from jax.experimental.pallas import tpu_sc as plsc

N = 10000
C = 128
K = 27
EK = 12000
E = K * EK
EPS = 1e-5

NUM_CORES = 2
NUM_SUBCORES = 16
WIN = 120
NWIN = E // WIN
NWIN_PER_CORE = NWIN // NUM_CORES
MAX_WIN_PER_SUB = -(-NWIN_PER_CORE // NUM_SUBCORES)
ROWS_UNIF = 624
TAIL_OFF = ROWS_UNIF * NUM_SUBCORES
TAIL = N - TAIL_OFF

MM_BLK = 1000
NB = N // MM_BLK
KG = 9
NKG = K // KG


def _mm_body(x_ref, w_ref, z_ref):
    res = jnp.dot(x_ref[...], w_ref[...], preferred_element_type=jnp.float32)
    for t in range(KG):
        z_ref[t] = res[:, t * C:(t + 1) * C]


def _z_from_x(x, wr):
    return pl.pallas_call(
        _mm_body,
        grid=(NB, NKG),
        in_specs=[
            pl.BlockSpec((MM_BLK, C), lambda i, g: (i, 0)),
            pl.BlockSpec((C, KG * C), lambda i, g: (0, g)),
        ],
        out_specs=pl.BlockSpec((KG, MM_BLK, C), lambda i, g: (g, i, 0)),
        out_shape=jax.ShapeDtypeStruct((K, N, C), jnp.float32),
    )(x, wr)


def _mm_norm_body(p_ref, sc_ref, sh_ref, w_ref, z_ref):
    h = (p_ref[0] + p_ref[1]) * sc_ref[...] + sh_ref[...]
    h = jnp.maximum(h, 0.0)
    res = jnp.dot(h, w_ref[...], preferred_element_type=jnp.float32)
    for t in range(KG):
        z_ref[t] = res[:, t * C:(t + 1) * C]


def _z_from_partials(p, scale, shift, wr):
    return pl.pallas_call(
        _mm_norm_body,
        grid=(NB, NKG),
        in_specs=[
            pl.BlockSpec((NUM_CORES, MM_BLK, C), lambda i, g: (0, i, 0)),
            pl.BlockSpec((1, C), lambda i, g: (0, 0)),
            pl.BlockSpec((1, C), lambda i, g: (0, 0)),
            pl.BlockSpec((C, KG * C), lambda i, g: (0, g)),
        ],
        out_specs=pl.BlockSpec((KG, MM_BLK, C), lambda i, g: (g, i, 0)),
        out_shape=jax.ShapeDtypeStruct((K, N, C), jnp.float32),
    )(p, scale, shift, wr)


def _stats_body(p_ref, sum_ref, sq_ref):
    i = pl.program_id(0)

    @pl.when(i == 0)
    def _():
        sum_ref[...] = jnp.zeros_like(sum_ref)
        sq_ref[...] = jnp.zeros_like(sq_ref)

    h = p_ref[0] + p_ref[1]
    sum_ref[...] += jnp.sum(h, axis=0, keepdims=True)
    sq_ref[...] += jnp.sum(h * h, axis=0, keepdims=True)


def _stats(p):
    return pl.pallas_call(
        _stats_body,
        grid=(NB,),
        in_specs=[pl.BlockSpec((NUM_CORES, MM_BLK, C), lambda i: (0, i, 0))],
        out_specs=[
            pl.BlockSpec((1, C), lambda i: (0, 0)),
            pl.BlockSpec((1, C), lambda i: (0, 0)),
        ],
        out_shape=[
            jax.ShapeDtypeStruct((1, C), jnp.float32),
            jax.ShapeDtypeStruct((1, C), jnp.float32),
        ],
    )(p)


def _final_body(p_ref, sc_ref, sh_ref, x_ref, o_ref):
    h = (p_ref[0] + p_ref[1]) * sc_ref[...] + sh_ref[...] + x_ref[...]
    o_ref[...] = jnp.maximum(h, 0.0)


def _final(p, scale, shift, x):
    return pl.pallas_call(
        _final_body,
        grid=(NB,),
        in_specs=[
            pl.BlockSpec((NUM_CORES, MM_BLK, C), lambda i: (0, i, 0)),
            pl.BlockSpec((1, C), lambda i: (0, 0)),
            pl.BlockSpec((1, C), lambda i: (0, 0)),
            pl.BlockSpec((MM_BLK, C), lambda i: (i, 0)),
        ],
        out_specs=pl.BlockSpec((MM_BLK, C), lambda i: (i, 0)),
        out_shape=jax.ShapeDtypeStruct((N, C), jnp.float32),
    )(p, scale, shift, x)


NG = (MAX_WIN_PER_SUB + 1) // 2
SRC_LOC = MAX_WIN_PER_SUB * WIN


@functools.partial(
    pl.kernel,
    out_type=jax.ShapeDtypeStruct((NUM_CORES, N, C), jnp.float32),
    mesh=plsc.VectorSubcoreMesh(core_axis_name="c", subcore_axis_name="s"),
    scratch_types=[
        pltpu.VMEM((SRC_LOC,), jnp.int32),
        pltpu.VMEM((WIN,), jnp.int32),
        pltpu.VMEM((WIN,), jnp.int32),
        pltpu.VMEM((WIN, C), jnp.float32),
        pltpu.VMEM((WIN, C), jnp.float32),
        pltpu.VMEM_SHARED((N, C), jnp.float32),
        pltpu.SemaphoreType.DMA,
        pltpu.SemaphoreType.DMA,
        pltpu.SemaphoreType.DMA,
        pltpu.SemaphoreType.DMA,
        pltpu.SemaphoreType.DMA,
        pltpu.SemaphoreType.DMA,
    ],
)
def _edge_conv(z_hbm, src_hbm, dst_hbm, zero_hbm, out_hbm,
               src_loc, dstv0, dstv1, rows0, rows1, acc,
               isem0, isem1, gsem0, gsem1, ssem0, ssem1):
    c = lax.axis_index("c")
    s = lax.axis_index("s")

    start_win = c * NWIN_PER_CORE + s * 84 + jnp.minimum(s, 6)
    cnt = jnp.where(s < 6, 85, 84)
    bulk_start = jnp.minimum(start_win, NWIN - MAX_WIN_PER_SUB)
    loc_off = start_win - bulk_start

    pltpu.sync_copy(src_hbm.at[pl.ds(bulk_start * WIN, SRC_LOC)], src_loc)

    pltpu.sync_copy(zero_hbm.at[pl.ds(s * ROWS_UNIF, ROWS_UNIF)],
                    acc.at[pl.ds(s * ROWS_UNIF, ROWS_UNIF)])

    @pl.when(s == 0)
    def _():
        pltpu.sync_copy(zero_hbm.at[pl.ds(TAIL_OFF, TAIL)],
                        acc.at[pl.ds(TAIL_OFF, TAIL)])

    plsc.subcore_barrier()

    def starts(dstv, rows, isem, gsem, j):
        w = start_win + j
        k = w // (EK // WIN)
        pltpu.async_copy(dst_hbm.at[pl.ds(w * WIN, WIN)], dstv, isem)
        pltpu.async_copy(
            z_hbm.at[k].at[src_loc.at[pl.ds((j + loc_off) * WIN, WIN)]],
            rows, gsem)

    def fire(dstv, rows, isem, gsem, j):
        @pl.when(j < cnt)
        def _():
            starts(dstv, rows, isem, gsem, j)

    def fire_next(dstv, rows, isem, gsem, ssem, j):
        @pl.when(j < cnt)
        def _():
            pltpu.make_async_copy(rows, acc.at[dstv], ssem).wait()
            starts(dstv, rows, isem, gsem, j)

    def drain(dstv, rows, isem, gsem, ssem, j):
        @pl.when(j < cnt)
        def _():
            w = start_win + j
            k = w // (EK // WIN)
            pltpu.make_async_copy(dst_hbm.at[pl.ds(w * WIN, WIN)],
                                  dstv, isem).wait()
            pltpu.make_async_copy(
                z_hbm.at[k].at[src_loc.at[pl.ds((j + loc_off) * WIN, WIN)]],
                rows, gsem).wait()
            pltpu.async_copy(rows, acc.at[dstv], ssem, add=True)

    fire(dstv0, rows0, isem0, gsem0, 0)
    fire(dstv1, rows1, isem1, gsem1, 1)

    @pl.loop(0, NG)
    def _(g):
        j0 = 2 * g
        drain(dstv0, rows0, isem0, gsem0, ssem0, j0)
        fire_next(dstv0, rows0, isem0, gsem0, ssem0, j0 + 2)
        drain(dstv1, rows1, isem1, gsem1, ssem1, j0 + 1)
        fire_next(dstv1, rows1, isem1, gsem1, ssem1, j0 + 3)

    pltpu.make_async_copy(rows0, acc.at[dstv0], ssem0).wait()
    pltpu.make_async_copy(rows1, acc.at[dstv1], ssem1).wait()

    plsc.subcore_barrier()
    pltpu.sync_copy(acc.at[pl.ds(s * ROWS_UNIF, ROWS_UNIF)],
                    out_hbm.at[c].at[pl.ds(s * ROWS_UNIF, ROWS_UNIF)])

    @pl.when(s == 0)
    def _():
        pltpu.sync_copy(acc.at[pl.ds(TAIL_OFF, TAIL)],
                        out_hbm.at[c].at[pl.ds(TAIL_OFF, TAIL)])


def _bn_coeffs(s, q, g, b):
    mu = s / N
    var = q / N - mu * mu
    rs = g.reshape(1, C) / jnp.sqrt(var + EPS)
    return rs, b.reshape(1, C) - mu * rs


def kernel(x, edge_index, W1, g1, b1, W2, g2, b2):
    src = edge_index[0]
    dst = edge_index[1]
    zeros = jnp.zeros((N, C), jnp.float32)
    w1r = jnp.transpose(W1, (1, 0, 2)).reshape(C, K * C)
    w2r = jnp.transpose(W2, (1, 0, 2)).reshape(C, K * C)

    z1 = _z_from_x(x, w1r)
    p1 = _edge_conv(z1, src, dst, zeros)
    s1, q1 = _stats(p1)
    scale1, shift1 = _bn_coeffs(s1, q1, g1, b1)

    z2 = _z_from_partials(p1, scale1, shift1, w2r)
    p2 = _edge_conv(z2, src, dst, zeros)
    s2, q2 = _stats(p2)
    scale2, shift2 = _bn_coeffs(s2, q2, g2, b2)

    return _final(p2, scale2, shift2, x)
